# baseline (speedup 1.0000x reference)
import functools

import jax
import jax.numpy as jnp
from jax import lax
from jax.experimental import pallas as pl
from jax.experimental.pallas import tpu as pltpu
from jax.experimental.pallas import tpu_sc as plsc

_NC = 2
_NS = 16
_NW = _NC * _NS
_LANES = 16

_NBUF = 4
_UNROLL = 8


SPAN = 2


def _make_transpose(V, EMB):
  ntc = V // 128
  tail = V - ntc * 128
  nsp = ntc // SPAN
  assert nsp * SPAN == ntc
  per_w = -(-nsp // _NW)
  w = 128 * SPAN
  mesh = plsc.VectorSubcoreMesh(core_axis_name="c", subcore_axis_name="s")

  @functools.partial(
      pl.kernel,
      mesh=mesh,
      out_type=jax.ShapeDtypeStruct((V * EMB,), jnp.float32),
      scratch_types=[
          [pltpu.VMEM((EMB, w), jnp.float32) for _ in range(2)],
          [pltpu.VMEM((w * EMB,), jnp.float32) for _ in range(2)],
          [pltpu.VMEM_SHARED((_NS, EMB, w), jnp.float32) for _ in range(2)],
          [pltpu.VMEM_SHARED((_NS, w * EMB), jnp.float32) for _ in range(2)],
          [pltpu.SemaphoreType.DMA for _ in range(2)],
          [pltpu.SemaphoreType.DMA for _ in range(2)],
      ],
      compiler_params=pltpu.CompilerParams(use_tc_tiling_on_sc=True,
                                           needs_layout_passes=False),
  )
  def transpose(tabT_hbm, tail_hbm, pair_hbm, pins, tbs, sins, souts, semi,
                semo):
    wid = lax.axis_index("s") * _NC + lax.axis_index("c")
    sid = lax.axis_index("s")
    iota = lax.iota(jnp.int32, _LANES)
    start = wid * per_w
    cnt = jnp.minimum(per_w, jnp.maximum(nsp - start, 0))
    flats = [(iota + 16 * j) * EMB for j in range(w // 16)]

    def issue_in(k, b):
      off = pl.multiple_of((start + k) * w, 128)
      for te in range(EMB // 8):
        pltpu.async_copy(tabT_hbm.at[pl.ds(8 * te, 8), pl.ds(off, w)],
                         sins[b].at[sid, pl.ds(8 * te, 8)], semi[b])

    def drain_in(k, b):
      off = pl.multiple_of((start + k) * w, 128)
      pltpu.make_async_copy(tabT_hbm.at[:, pl.ds(off, w)],
                            sins[b].at[sid], semi[b]).wait()

    def out_slice(k):
      return pair_hbm.at[pl.ds(pl.multiple_of((start + k) * w * EMB, 8),
                               w * EMB)]

    for p in range(2):

      @pl.when(cnt > p)
      def _():
        issue_in(p, p)

    def p1_body(k, carry):
      for b in range(2):
        i = k * 2 + b

        @pl.when(i < cnt)
        def _():
          drain_in(i, b)
          pltpu.sync_copy(sins[b].at[sid], pins[b])

          @pl.when(i >= 2)
          def _():
            pltpu.make_async_copy(souts[b].at[sid], out_slice(i - 2),
                                  semo[b]).wait()

          def erow(e, carry2):
            ev = jnp.full((_LANES,), e, jnp.int32)
            for j0 in range(0, w // 16, 8):
              vals = [pins[b][e, pl.ds(16 * (j0 + j), 16)] for j in range(8)]
              addrs = [flats[j0 + j] + ev for j in range(8)]
              for j in range(8):
                plsc.store_scatter(tbs[b], [addrs[j]], vals[j])
            return carry2

          lax.fori_loop(0, EMB, erow, 0)
          pltpu.sync_copy(tbs[b], souts[b].at[sid])
          pltpu.async_copy(souts[b].at[sid], out_slice(i), semo[b])
          nk = i + 2

          @pl.when(nk < cnt)
          def _():
            issue_in(nk, b)

      return carry

    lax.fori_loop(0, (per_w + 1) // 2, p1_body, 0)

    for b in range(2):
      last = cnt - 2 + b

      @pl.when(last >= 0)
      def _():
        pltpu.make_async_copy(souts[b].at[sid], out_slice(last),
                              semo[b]).wait()

    @pl.when(wid == _NW - 1)
    def _():
      pltpu.sync_copy(tail_hbm, tbs[0].at[pl.ds(0, tail * EMB)])
      pltpu.sync_copy(tbs[0].at[pl.ds(0, tail * EMB)],
                      pair_hbm.at[pl.ds(ntc * 128 * EMB, tail * EMB)])

  return transpose


def _make_gather_sum(B, L, V, EMB):
  assert B % _NW == 0 and EMB % _LANES == 0
  bpw = B // _NW
  chunk = L // 2
  assert chunk * 2 == L and chunk <= 128
  assert L % _UNROLL == 0 and bpw % _NBUF == 0
  nvec = EMB // _LANES
  mesh = plsc.VectorSubcoreMesh(core_axis_name="c", subcore_axis_name="s")

  @functools.partial(
      pl.kernel,
      mesh=mesh,
      out_type=jax.ShapeDtypeStruct((B, EMB), jnp.float32),
      scratch_types=[
          pltpu.VMEM((bpw, 2, chunk), jnp.int32),
          [pltpu.VMEM((L, EMB), jnp.float32) for _ in range(_NBUF)],
          pltpu.VMEM((bpw, EMB), jnp.float32),
          [pltpu.SemaphoreType.DMA for _ in range(_NBUF)],
      ],
      compiler_params=pltpu.CompilerParams(use_tc_tiling_on_sc=False),
  )
  def gather_sum(idx_hbm, table_hbm, sums_hbm, idx_v, bufs, outv, sems):
    wid = lax.axis_index("s") * _NC + lax.axis_index("c")
    base = wid * bpw
    pltpu.sync_copy(idx_hbm.at[pl.ds(base, bpw)], idx_v)

    def issue(j, b):
      pltpu.async_copy(table_hbm.at[idx_v.at[j, 0]],
                       bufs[b].at[pl.ds(0, chunk)], sems[b])
      pltpu.async_copy(table_hbm.at[idx_v.at[j, 1]],
                       bufs[b].at[pl.ds(chunk, chunk)], sems[b])

    def drain(j, b):
      pltpu.make_async_copy(table_hbm.at[idx_v.at[j, 0]],
                            bufs[b].at[pl.ds(0, chunk)], sems[b]).wait()
      pltpu.make_async_copy(table_hbm.at[idx_v.at[j, 1]],
                            bufs[b].at[pl.ds(chunk, chunk)], sems[b]).wait()

    for b in range(_NBUF):
      issue(b, b)

    zero = jnp.zeros((_LANES,), jnp.float32)

    def group(g, carry):
      for b in range(_NBUF):
        i = g * _NBUF + b
        drain(i, b)
        buf = bufs[b]

        def red(r, acc):
          rb = r * _UNROLL
          nxt = []
          for k in range(nvec):
            t = [buf[rb + d, pl.ds(_LANES * k, _LANES)]
                 for d in range(_UNROLL)]
            while len(t) > 1:
              t = [t[p] + t[p + 1] for p in range(0, len(t), 2)]
            nxt.append(acc[k] + t[0])
          return tuple(nxt)

        acc = lax.fori_loop(0, L // _UNROLL, red, (zero,) * nvec)
        for k in range(nvec):
          outv[i, pl.ds(_LANES * k, _LANES)] = acc[k]

        nj = i + _NBUF

        @pl.when(nj < bpw)
        def _():
          issue(nj, b)

      return carry

    lax.fori_loop(0, bpw // _NBUF, group, 0)
    pltpu.sync_copy(outv, sums_hbm.at[pl.ds(base, bpw)])

  return gather_sum


def _make_dense(TB, B, EMB, IMG, HID, blk=512):
  assert TB % blk == 0 and B % blk == 0
  grid = (TB // blk,)
  nrep = B // blk

  def body(sums_ref, img_ref, w1e_ref, w1i_ref, b1_ref, w2_ref, b2_ref,
           out_ref):
    x = jnp.dot(sums_ref[...], w1e_ref[...],
                preferred_element_type=jnp.float32)
    x = x + jnp.dot(img_ref[...], w1i_ref[...],
                    preferred_element_type=jnp.float32)
    h = jnp.tanh(x + b1_ref[...])
    o = jnp.sum(h * w2_ref[...], axis=1) + b2_ref[0, 0]
    out_ref[...] = o.reshape(1, blk)

  return pl.pallas_call(
      body,
      grid=grid,
      in_specs=[
          pl.BlockSpec((blk, EMB), lambda g: (g % nrep, 0)),
          pl.BlockSpec((blk, IMG), lambda g: (g, 0)),
          pl.BlockSpec((EMB, HID), lambda g: (0, 0)),
          pl.BlockSpec((IMG, HID), lambda g: (0, 0)),
          pl.BlockSpec((1, HID), lambda g: (0, 0)),
          pl.BlockSpec((1, HID), lambda g: (0, 0)),
          pl.BlockSpec((1, 1), lambda g: (0, 0), memory_space=pltpu.SMEM),
      ],
      out_specs=pl.BlockSpec((1, blk), lambda g: (0, g)),
      out_shape=jax.ShapeDtypeStruct((1, TB), jnp.float32),
      compiler_params=pltpu.CompilerParams(
          dimension_semantics=("arbitrary",)),
  )


@jax.jit
def kernel(inputs, img_feat, emb_table, W1, b1, W2, b2):
  B, L = inputs.shape
  V, EMB = emb_table.shape
  TB, IMG = img_feat.shape
  HID = W1.shape[1]

  ntc = V // 128
  tail_flat = emb_table[ntc * 128:].reshape(-1)
  flat = _make_transpose(V, EMB)(emb_table.T, tail_flat)
  tab_lin = flat.reshape(V, EMB)

  idx3 = inputs.reshape(B, 2, L // 2)
  sums = _make_gather_sum(B, L, V, EMB)(idx3, tab_lin)

  dense = _make_dense(TB, B, EMB, IMG, HID)
  out = dense(sums, img_feat, W1[:EMB], W1[EMB:], b1.reshape(1, HID),
              W2.reshape(1, HID), b2.reshape(1, 1))
  return out


# speedup vs baseline: 2.0833x; 2.0833x over previous
import functools

import jax
import jax.numpy as jnp
from jax import lax
from jax.experimental import pallas as pl
from jax.experimental.pallas import tpu as pltpu
from jax.experimental.pallas import tpu_sc as plsc

_NC = 2
_NS = 16
_NW = _NC * _NS
_LANES = 16

_NBUF = 4
_UNROLL = 8


def _make_gather_sum(B, L, V, EMB):
  assert B % _NW == 0 and EMB % _LANES == 0
  bpw = B // _NW
  chunk = L // 2
  assert chunk * 2 == L and chunk <= 128
  assert L % _UNROLL == 0 and bpw % _NBUF == 0
  nvec = EMB // _LANES
  mesh = plsc.VectorSubcoreMesh(core_axis_name="c", subcore_axis_name="s")

  @functools.partial(
      pl.kernel,
      mesh=mesh,
      out_type=jax.ShapeDtypeStruct((B, EMB), jnp.float32),
      scratch_types=[
          pltpu.VMEM((bpw, 2, chunk), jnp.int32),
          [pltpu.VMEM((L, EMB), jnp.float32) for _ in range(_NBUF)],
          pltpu.VMEM((bpw, EMB), jnp.float32),
          [pltpu.SemaphoreType.DMA for _ in range(_NBUF)],
      ],
      compiler_params=pltpu.CompilerParams(use_tc_tiling_on_sc=False),
  )
  def gather_sum(idx_hbm, table_hbm, sums_hbm, idx_v, bufs, outv, sems):
    wid = lax.axis_index("s") * _NC + lax.axis_index("c")
    base = wid * bpw
    pltpu.sync_copy(idx_hbm.at[pl.ds(base, bpw)], idx_v)

    def issue(j, b):
      pltpu.async_copy(table_hbm.at[idx_v.at[j, 0]],
                       bufs[b].at[pl.ds(0, chunk)], sems[b])
      pltpu.async_copy(table_hbm.at[idx_v.at[j, 1]],
                       bufs[b].at[pl.ds(chunk, chunk)], sems[b])

    def drain(j, b):
      pltpu.make_async_copy(table_hbm.at[idx_v.at[j, 0]],
                            bufs[b].at[pl.ds(0, chunk)], sems[b]).wait()
      pltpu.make_async_copy(table_hbm.at[idx_v.at[j, 1]],
                            bufs[b].at[pl.ds(chunk, chunk)], sems[b]).wait()

    for b in range(_NBUF):
      issue(b, b)

    zero = jnp.zeros((_LANES,), jnp.float32)

    def group(g, carry):
      for b in range(_NBUF):
        i = g * _NBUF + b
        drain(i, b)
        buf = bufs[b]

        def red(r, acc):
          rb = r * _UNROLL
          nxt = []
          for k in range(nvec):
            t = [buf[rb + d, pl.ds(_LANES * k, _LANES)]
                 for d in range(_UNROLL)]
            while len(t) > 1:
              t = [t[p] + t[p + 1] for p in range(0, len(t), 2)]
            nxt.append(acc[k] + t[0])
          return tuple(nxt)

        acc = lax.fori_loop(0, L // _UNROLL, red, (zero,) * nvec)
        for k in range(nvec):
          outv[i, pl.ds(_LANES * k, _LANES)] = acc[k]

        nj = i + _NBUF

        @pl.when(nj < bpw)
        def _():
          issue(nj, b)

      return carry

    lax.fori_loop(0, bpw // _NBUF, group, 0)
    pltpu.sync_copy(outv, sums_hbm.at[pl.ds(base, bpw)])

  return gather_sum


def _make_dense(TB, B, EMB, IMG, HID, blk=512):
  assert TB % blk == 0 and B % blk == 0
  grid = (TB // blk,)
  nrep = B // blk

  def body(sums_ref, img_ref, w1e_ref, w1i_ref, b1_ref, w2_ref, b2_ref,
           out_ref):
    x = jnp.dot(sums_ref[...], w1e_ref[...],
                preferred_element_type=jnp.float32)
    x = x + jnp.dot(img_ref[...], w1i_ref[...],
                    preferred_element_type=jnp.float32)
    h = jnp.tanh(x + b1_ref[...])
    o = jnp.sum(h * w2_ref[...], axis=1) + b2_ref[0, 0]
    out_ref[...] = o.reshape(1, blk)

  return pl.pallas_call(
      body,
      grid=grid,
      in_specs=[
          pl.BlockSpec((blk, EMB), lambda g: (g % nrep, 0)),
          pl.BlockSpec((blk, IMG), lambda g: (g, 0)),
          pl.BlockSpec((EMB, HID), lambda g: (0, 0)),
          pl.BlockSpec((IMG, HID), lambda g: (0, 0)),
          pl.BlockSpec((1, HID), lambda g: (0, 0)),
          pl.BlockSpec((1, HID), lambda g: (0, 0)),
          pl.BlockSpec((1, 1), lambda g: (0, 0), memory_space=pltpu.SMEM),
      ],
      out_specs=pl.BlockSpec((1, blk), lambda g: (0, g)),
      out_shape=jax.ShapeDtypeStruct((1, TB), jnp.float32),
      compiler_params=pltpu.CompilerParams(
          dimension_semantics=("arbitrary",)),
  )


@jax.jit
def kernel(inputs, img_feat, emb_table, W1, b1, W2, b2):
  B, L = inputs.shape
  V, EMB = emb_table.shape
  TB, IMG = img_feat.shape
  HID = W1.shape[1]

  idx3 = inputs.reshape(B, 2, L // 2)
  sums = _make_gather_sum(B, L, V, EMB)(idx3, emb_table)

  dense = _make_dense(TB, B, EMB, IMG, HID)
  out = dense(sums, img_feat, W1[:EMB], W1[EMB:], b1.reshape(1, HID),
              W2.reshape(1, HID), b2.reshape(1, 1))
  return out
